# speedup vs baseline: 6.3900x; 6.3900x over previous
import functools

import jax
import jax.numpy as jnp
from jax import lax
from jax.experimental import pallas as pl
from jax.experimental.pallas import tpu as pltpu
from jax.experimental.pallas import tpu_sc as plsc

NC = 2
NS = 16
NW = NC * NS
LANES = 16

CHUNK = 400
SUB = 80
NSUB = CHUNK // SUB


def _sc_embed(idx, emb_weight, pos_table, seq):
    T = idx.shape[0]
    D = emb_weight.shape[1]
    per_tile = T // NW
    n_chunks = per_tile // CHUNK
    assert T == per_tile * NW and per_tile == n_chunks * CHUNK
    assert CHUNK % seq == 0 and per_tile % seq == 0

    mesh = plsc.VectorSubcoreMesh(core_axis_name="c", subcore_axis_name="s")

    @functools.partial(
        pl.kernel,
        out_type=jax.ShapeDtypeStruct((T, D), jnp.float32),
        mesh=mesh,
        scratch_types=[
            pltpu.VMEM((CHUNK,), jnp.int32),
            pltpu.VMEM((CHUNK,), jnp.int32),
            pltpu.VMEM((CHUNK, D), jnp.float32),
            pltpu.VMEM((CHUNK, D), jnp.float32),
            pltpu.SemaphoreType.DMA,
            pltpu.SemaphoreType.DMA,
        ],
    )
    def body(idx_hbm, tab_hbm, ptab_hbm, out_hbm, idx_v, pos_v, tok_r, pos_r,
             sem_t, sem_p):
        wid = lax.axis_index("s") * NC + lax.axis_index("c")
        base = wid * per_tile

        def chunk_body(c, carry):
            tbase = base + c * CHUNK
            pltpu.sync_copy(idx_hbm.at[pl.ds(tbase, CHUNK)], idx_v)

            def pos_body(j, carry2):
                t = idx_v[pl.ds(j * LANES, LANES)]
                v = lax.iota(jnp.int32, LANES) + j * LANES
                v = jnp.where(v >= seq, v - seq, v) + 1
                pos_v[pl.ds(j * LANES, LANES)] = jnp.where(t == 0, 0, v)
                return carry2

            lax.fori_loop(0, CHUNK // LANES, pos_body, 0)

            cps = []
            for k in range(NSUB):
                rows = pl.ds(k * SUB, SUB)
                cps.append(pltpu.async_copy(
                    tab_hbm.at[idx_v.at[rows]], tok_r.at[rows], sem_t))
                cps.append(pltpu.async_copy(
                    ptab_hbm.at[pos_v.at[rows]], pos_r.at[rows], sem_p))
            for cp in cps:
                cp.wait()

            def add_body(i, carry2):
                for jj in range(D // LANES):
                    cols = pl.ds(jj * LANES, LANES)
                    tok_r[i, cols] = tok_r[i, cols] + pos_r[i, cols]
                return carry2

            lax.fori_loop(0, CHUNK, add_body, 0)

            pltpu.sync_copy(tok_r, out_hbm.at[pl.ds(tbase, CHUNK)])
            return carry

        lax.fori_loop(0, n_chunks, chunk_body, 0)

    return body(idx, emb_weight, pos_table)


def kernel(inputs, emb_weight, pos_table):
    B, L = inputs.shape
    D = emb_weight.shape[1]
    out = _sc_embed(inputs.reshape(B * L), emb_weight, pos_table, L)
    return out.reshape(B, L, D)
